# baseline (speedup 1.0000x reference)
import functools

import jax
import jax.numpy as jnp
from jax import lax
from jax.experimental import pallas as pl
from jax.experimental.pallas import tpu as pltpu
from jax.experimental.pallas import tpu_sc as plsc

_VOCAB = 30522
_H = 768
_MAXP = 512
_B = 64
_S = 512
_EPS = 1e-5

_NC = 2
_NS = 16
_NW = _NC * _NS
_PPW = _S // _NW
_L = 16
_NVR = _H // _L


def _lane_total(x, tmp):
    tmp[pl.ds(16, _L)] = jnp.zeros((_L,), jnp.float32)
    y = x
    for sh in (8, 4, 2, 1):
        tmp[pl.ds(0, _L)] = y
        y = y + tmp[pl.ds(sh, _L)]
    return jnp.broadcast_to(y[0], (_L,))


def _body(ids_hbm, ww_hbm, wp_hbm, wt_hbm, g_hbm, bt_hbm, out_hbm,
          ids_v, pos_v, wt_v, g_v, bt_v, rows_v, tmp_v, sem):
    wid = lax.axis_index("s") * _NC + lax.axis_index("c")
    p0 = wid * _PPW

    pltpu.sync_copy(ids_hbm, ids_v)
    pltpu.sync_copy(wp_hbm.at[pl.ds(p0, _PPW)], pos_v)
    pltpu.sync_copy(wt_hbm, wt_v)
    pltpu.sync_copy(g_hbm, g_v)
    pltpu.sync_copy(bt_hbm, bt_v)

    for t in range(_PPW):
        def fold(j, _):
            sl = pl.ds(j * _L, _L)
            pos_v[t, sl] = pos_v[t, sl] + wt_v[0, sl]
            return 0
        lax.fori_loop(0, _NVR, fold, 0)

    inv_h = 1.0 / _H

    def do_batch(b, _):
        idx = ids_v[b, pl.ds(p0, _PPW)]
        pltpu.async_copy(ww_hbm.at[idx], rows_v, sem).wait()

        for t in range(_PPW):
            def p1(j, carry):
                acc, acc2 = carry
                sl = pl.ds(j * _L, _L)
                x = rows_v[t, sl] + pos_v[t, sl]
                rows_v[t, sl] = x
                return acc + x, acc2 + x * x

            z = jnp.zeros((_L,), jnp.float32)
            acc, acc2 = lax.fori_loop(0, _NVR, p1, (z, z))

            mean = _lane_total(acc, tmp_v) * inv_h
            ex2 = _lane_total(acc2, tmp_v) * inv_h
            v = ex2 - mean * mean + _EPS
            iv = lax.bitcast_convert_type(v, jnp.int32)
            iv = 0x5F3759DF - lax.shift_right_logical(iv, 1)
            y = lax.bitcast_convert_type(iv, jnp.float32)
            y = y * (1.5 - 0.5 * v * y * y)
            y = y * (1.5 - 0.5 * v * y * y)
            y = y * (1.5 - 0.5 * v * y * y)

            def p2(j, _c):
                sl = pl.ds(j * _L, _L)
                x = rows_v[t, sl]
                rows_v[t, sl] = (x - mean) * (y * g_v[sl]) + bt_v[sl]
                return 0

            lax.fori_loop(0, _NVR, p2, 0)

        pltpu.sync_copy(rows_v, out_hbm.at[pl.ds(b * _S + p0, _PPW)])
        return 0

    lax.fori_loop(0, _B, do_batch, 0)


@jax.jit
def _launch(ids, ww, wp, wt, g, bt):
    mesh = plsc.VectorSubcoreMesh(core_axis_name="c", subcore_axis_name="s")
    run = functools.partial(
        pl.kernel,
        out_type=jax.ShapeDtypeStruct((_B * _S, _H), jnp.float32),
        mesh=mesh,
        scratch_types=[
            pltpu.VMEM((_B, _S), jnp.int32),
            pltpu.VMEM((_PPW, _H), jnp.float32),
            pltpu.VMEM((2, _H), jnp.float32),
            pltpu.VMEM((_H,), jnp.float32),
            pltpu.VMEM((_H,), jnp.float32),
            pltpu.VMEM((_PPW, _H), jnp.float32),
            pltpu.VMEM((2 * _L,), jnp.float32),
            pltpu.SemaphoreType.DMA,
        ],
    )(_body)
    return run(ids, ww, wp, wt, g, bt)


def kernel(input_ids, W_word, W_pos, W_type, gamma, beta):
    ids = input_ids.astype(jnp.int32)
    out = _launch(ids, W_word, W_pos, W_type, gamma, beta)
    return out.reshape(_B, _S, _H)


# speedup vs baseline: 1.6993x; 1.6993x over previous
import functools

import jax
import jax.numpy as jnp
from jax import lax
from jax.experimental import pallas as pl
from jax.experimental.pallas import tpu as pltpu
from jax.experimental.pallas import tpu_sc as plsc

_VOCAB = 30522
_H = 768
_B = 64
_S = 512
_EPS = 1e-5

_NC = 2
_NS = 16
_NW = _NC * _NS
_PPW = _S // _NW
_L = 16
_NVR = _H // _L


def _body(ids_hbm, ww_hbm, wp_hbm, wt_hbm, g_hbm, bt_hbm, out_hbm,
          ids_v, pos_v, wt_v, g_v, bt_v, rows_a, rows_b,
          sums_v, sq_v, m_sto, r_sto,
          gsem_a, gsem_b, osem_a, osem_b):
    wid = lax.axis_index("s") * _NC + lax.axis_index("c")
    p0 = wid * _PPW

    pltpu.sync_copy(ids_hbm, ids_v)
    pltpu.sync_copy(wp_hbm.at[pl.ds(p0, _PPW)], pos_v)
    pltpu.sync_copy(wt_hbm, wt_v)
    pltpu.sync_copy(g_hbm, g_v)
    pltpu.sync_copy(bt_hbm, bt_v)

    def fold(t, _):
        def fj(j, _c):
            sl = pl.ds(j * _L, _L)
            pos_v[t, sl] = pos_v[t, sl] + wt_v[0, sl]
            return 0
        return lax.fori_loop(0, _NVR, fj, 0)

    lax.fori_loop(0, _PPW, fold, 0)

    inv_h = 1.0 / _H
    lane_iota = lax.iota(jnp.int32, _L)
    zero = jnp.zeros((_L,), jnp.float32)

    def start_gather(b, buf, sem):
        idx = ids_v[b, pl.ds(p0, _PPW)]
        pltpu.async_copy(ww_hbm.at[idx], buf, sem)

    def wait_gather(buf, sem):
        pltpu.make_async_copy(ww_hbm.at[pl.ds(0, _PPW)], buf, sem).wait()

    def start_out(b, buf, sem):
        pltpu.async_copy(buf, out_hbm.at[pl.ds(b * _S + p0, _PPW)], sem)

    def wait_out(buf, sem):
        pltpu.make_async_copy(buf, out_hbm.at[pl.ds(0, _PPW)], sem).wait()

    def compute(buf):
        def p1(t, _):
            col = jnp.broadcast_to(t, (_L,))
            accs = [zero] * 4
            acc2s = [zero] * 4
            for j in range(_NVR):
                sl = pl.ds(j * _L, _L)
                x = buf[t, sl] + pos_v[t, sl]
                buf[t, sl] = x
                accs[j % 4] = accs[j % 4] + x
                acc2s[j % 4] = acc2s[j % 4] + x * x
            acc = (accs[0] + accs[1]) + (accs[2] + accs[3])
            acc2 = (acc2s[0] + acc2s[1]) + (acc2s[2] + acc2s[3])
            plsc.store_scatter(sums_v, [lane_iota, col], acc)
            plsc.store_scatter(sq_v, [lane_iota, col], acc2)
            return 0

        lax.fori_loop(0, _PPW, p1, 0)

        t1 = [zero] * 4
        t2 = [zero] * 4
        for i in range(_L):
            sl = pl.ds(0, _L)
            t1[i % 4] = t1[i % 4] + sums_v[i, sl]
            t2[i % 4] = t2[i % 4] + sq_v[i, sl]
        tot = (t1[0] + t1[1]) + (t1[2] + t1[3])
        tot2 = (t2[0] + t2[1]) + (t2[2] + t2[3])
        mean_vec = tot * inv_h
        var = tot2 * inv_h - mean_vec * mean_vec + _EPS
        iv = lax.bitcast_convert_type(var, jnp.int32)
        iv = 0x5F3759DF - lax.shift_right_logical(iv, 1)
        y = lax.bitcast_convert_type(iv, jnp.float32)
        y = y * (1.5 - 0.5 * var * y * y)
        y = y * (1.5 - 0.5 * var * y * y)
        y = y * (1.5 - 0.5 * var * y * y)
        m_sto[pl.ds(0, _L)] = mean_vec
        r_sto[pl.ds(0, _L)] = y

        def p2(t, _):
            tt = jnp.broadcast_to(t, (_L,))
            mean = plsc.load_gather(m_sto, [tt])
            rstd = plsc.load_gather(r_sto, [tt])
            for j in range(_NVR):
                sl = pl.ds(j * _L, _L)
                x = buf[t, sl]
                buf[t, sl] = (x - mean) * (rstd * g_v[sl]) + bt_v[sl]
            return 0

        lax.fori_loop(0, _PPW, p2, 0)

    start_gather(0, rows_a, gsem_a)

    bufs = ((rows_a, gsem_a, osem_a), (rows_b, gsem_b, osem_b))

    def outer(i2, _):
        for k in range(2):
            b = i2 * 2 + k
            buf, gsem, osem = bufs[k]
            obuf, ogsem, oosem = bufs[1 - k]
            if k == 0:
                @pl.when(i2 > 0)
                def _():
                    wait_out(obuf, oosem)
                start_gather(b + 1, obuf, ogsem)
            else:
                wait_out(obuf, oosem)

                @pl.when(i2 < _B // 2 - 1)
                def _():
                    start_gather(b + 1, obuf, ogsem)
            wait_gather(buf, gsem)
            compute(buf)
            start_out(b, buf, osem)
        return 0

    lax.fori_loop(0, _B // 2, outer, 0)
    wait_out(rows_b, osem_b)


@jax.jit
def _launch(ids, ww, wp, wt, g, bt):
    mesh = plsc.VectorSubcoreMesh(core_axis_name="c", subcore_axis_name="s")
    run = functools.partial(
        pl.kernel,
        out_type=jax.ShapeDtypeStruct((_B * _S, _H), jnp.float32),
        mesh=mesh,
        compiler_params=pltpu.CompilerParams(needs_layout_passes=False),
        scratch_types=[
            pltpu.VMEM((_B, _S), jnp.int32),
            pltpu.VMEM((_PPW, _H), jnp.float32),
            pltpu.VMEM((2, _H), jnp.float32),
            pltpu.VMEM((_H,), jnp.float32),
            pltpu.VMEM((_H,), jnp.float32),
            pltpu.VMEM((_PPW, _H), jnp.float32),
            pltpu.VMEM((_PPW, _H), jnp.float32),
            pltpu.VMEM((_L, _PPW), jnp.float32),
            pltpu.VMEM((_L, _PPW), jnp.float32),
            pltpu.VMEM((_L,), jnp.float32),
            pltpu.VMEM((_L,), jnp.float32),
            pltpu.SemaphoreType.DMA,
            pltpu.SemaphoreType.DMA,
            pltpu.SemaphoreType.DMA,
            pltpu.SemaphoreType.DMA,
        ],
    )(_body)
    return run(ids, ww, wp, wt, g, bt)


def kernel(input_ids, W_word, W_pos, W_type, gamma, beta):
    ids = input_ids.astype(jnp.int32)
    out = _launch(ids, W_word, W_pos, W_type, gamma, beta)
    return out.reshape(_B, _S, _H)


# speedup vs baseline: 3.3764x; 1.9869x over previous
import functools

import jax
import jax.numpy as jnp
from jax import lax
from jax.experimental import pallas as pl
from jax.experimental.pallas import tpu as pltpu
from jax.experimental.pallas import tpu_sc as plsc

_VOCAB = 30522
_H = 768
_B = 64
_S = 512
_EPS = 1e-5

_NC = 2
_NS = 16
_NW = _NC * _NS
_PPW = _S // _NW
_L = 16
_NVR = _H // _L


def _body(ids_hbm, ww_hbm, wp_hbm, wt_hbm, g_hbm, bt_hbm, out_hbm,
          ids_v, pos_v, wt_v, g_v, bt_v, rows_a, rows_b,
          sums_v, sq_v,
          gsem_a, gsem_b, osem_a, osem_b):
    wid = lax.axis_index("s") * _NC + lax.axis_index("c")
    p0 = wid * _PPW

    pltpu.sync_copy(ids_hbm, ids_v)
    pltpu.sync_copy(wp_hbm.at[pl.ds(p0, _PPW)], pos_v)
    pltpu.sync_copy(wt_hbm, wt_v)
    pltpu.sync_copy(g_hbm, g_v)
    pltpu.sync_copy(bt_hbm, bt_v)

    def fold(t, _):
        def fj(j, _c):
            sl = pl.ds(j * _L, _L)
            pos_v[t, sl] = pos_v[t, sl] + wt_v[0, sl]
            return 0
        return lax.fori_loop(0, _NVR, fj, 0)

    lax.fori_loop(0, _PPW, fold, 0)

    inv_h = 1.0 / _H
    lane_iota = lax.iota(jnp.int32, _L)
    zero = jnp.zeros((_L,), jnp.float32)

    def start_gather(b, buf, sem):
        idx = ids_v[b, pl.ds(p0, _PPW)]
        pltpu.async_copy(ww_hbm.at[idx], buf, sem)

    def wait_gather(buf, sem):
        pltpu.make_async_copy(ww_hbm.at[pl.ds(0, _PPW)], buf, sem).wait()

    def start_out(b, buf, sem):
        pltpu.async_copy(buf, out_hbm.at[pl.ds(b * _S + p0, _PPW)], sem)

    def wait_out(buf, sem):
        pltpu.make_async_copy(buf, out_hbm.at[pl.ds(0, _PPW)], sem).wait()

    def compute(buf):
        def p1(t, _):
            col = jnp.broadcast_to(t, (_L,))
            accs = [zero] * 4
            acc2s = [zero] * 4
            for j in range(_NVR):
                sl = pl.ds(j * _L, _L)
                x = buf[t, sl] + pos_v[t, sl]
                buf[t, sl] = x
                accs[j % 4] = accs[j % 4] + x
                acc2s[j % 4] = acc2s[j % 4] + x * x
            acc = (accs[0] + accs[1]) + (accs[2] + accs[3])
            acc2 = (acc2s[0] + acc2s[1]) + (acc2s[2] + acc2s[3])
            plsc.store_scatter(sums_v, [lane_iota, col], acc)
            plsc.store_scatter(sq_v, [lane_iota, col], acc2)
            return 0

        lax.fori_loop(0, _PPW, p1, 0)

        t1 = [zero] * 4
        t2 = [zero] * 4
        for i in range(_L):
            sl = pl.ds(0, _L)
            t1[i % 4] = t1[i % 4] + sums_v[i, sl]
            t2[i % 4] = t2[i % 4] + sq_v[i, sl]
        tot = (t1[0] + t1[1]) + (t1[2] + t1[3])
        tot2 = (t2[0] + t2[1]) + (t2[2] + t2[3])
        mean_vec = tot * inv_h
        var = tot2 * inv_h - mean_vec * mean_vec + _EPS
        iv = lax.bitcast_convert_type(var, jnp.int32)
        iv = 0x5F3759DF - lax.shift_right_logical(iv, 1)
        y = lax.bitcast_convert_type(iv, jnp.float32)
        y = y * (1.5 - 0.5 * var * y * y)
        y = y * (1.5 - 0.5 * var * y * y)
        y = y * (1.5 - 0.5 * var * y * y)

        for half in range(2):
            t0 = half * (_PPW // 2)
            means = [jnp.broadcast_to(mean_vec[t], (_L,))
                     for t in range(t0, t0 + _PPW // 2)]
            rstds = [jnp.broadcast_to(y[t], (_L,))
                     for t in range(t0, t0 + _PPW // 2)]

            def p2(j, _, t0=t0, means=means, rstds=rstds):
                sl = pl.ds(j * _L, _L)
                g = g_v[sl]
                bt = bt_v[sl]
                for i in range(_PPW // 2):
                    x = buf[t0 + i, sl]
                    buf[t0 + i, sl] = (x - means[i]) * (rstds[i] * g) + bt
                return 0

            lax.fori_loop(0, _NVR, p2, 0)

    start_gather(0, rows_a, gsem_a)

    bufs = ((rows_a, gsem_a, osem_a), (rows_b, gsem_b, osem_b))

    def outer(i2, _):
        for k in range(2):
            b = i2 * 2 + k
            buf, gsem, osem = bufs[k]
            obuf, ogsem, oosem = bufs[1 - k]
            if k == 0:
                @pl.when(i2 > 0)
                def _():
                    wait_out(obuf, oosem)
                start_gather(b + 1, obuf, ogsem)
            else:
                wait_out(obuf, oosem)

                @pl.when(i2 < _B // 2 - 1)
                def _():
                    start_gather(b + 1, obuf, ogsem)
            wait_gather(buf, gsem)
            compute(buf)
            start_out(b, buf, osem)
        return 0

    lax.fori_loop(0, _B // 2, outer, 0)
    wait_out(rows_b, osem_b)


@jax.jit
def _launch(ids, ww, wp, wt, g, bt):
    mesh = plsc.VectorSubcoreMesh(core_axis_name="c", subcore_axis_name="s")
    run = functools.partial(
        pl.kernel,
        out_type=jax.ShapeDtypeStruct((_B * _S, _H), jnp.float32),
        mesh=mesh,
        compiler_params=pltpu.CompilerParams(needs_layout_passes=False),
        scratch_types=[
            pltpu.VMEM((_B, _S), jnp.int32),
            pltpu.VMEM((_PPW, _H), jnp.float32),
            pltpu.VMEM((2, _H), jnp.float32),
            pltpu.VMEM((_H,), jnp.float32),
            pltpu.VMEM((_H,), jnp.float32),
            pltpu.VMEM((_PPW, _H), jnp.float32),
            pltpu.VMEM((_PPW, _H), jnp.float32),
            pltpu.VMEM((_L, _PPW), jnp.float32),
            pltpu.VMEM((_L, _PPW), jnp.float32),
            pltpu.SemaphoreType.DMA,
            pltpu.SemaphoreType.DMA,
            pltpu.SemaphoreType.DMA,
            pltpu.SemaphoreType.DMA,
        ],
    )(_body)
    return run(ids, ww, wp, wt, g, bt)


def kernel(input_ids, W_word, W_pos, W_type, gamma, beta):
    ids = input_ids.astype(jnp.int32)
    out = _launch(ids, W_word, W_pos, W_type, gamma, beta)
    return out.reshape(_B, _S, _H)


# speedup vs baseline: 3.4217x; 1.0134x over previous
import functools

import jax
import jax.numpy as jnp
from jax import lax
from jax.experimental import pallas as pl
from jax.experimental.pallas import tpu as pltpu
from jax.experimental.pallas import tpu_sc as plsc

_VOCAB = 30522
_H = 768
_B = 64
_S = 512
_EPS = 1e-5

_NC = 2
_NS = 16
_NW = _NC * _NS
_PPW = _S // _NW
_L = 16
_NVR = _H // _L


def _body(ids_hbm, ww_hbm, wp_hbm, wt_hbm, g_hbm, bt_hbm, out_hbm,
          ids_v, pos_v, wt_v, g_v, bt_v, rows_a, rows_b,
          sums_v, sq_v,
          gsem_a, gsem_b, osem_a, osem_b):
    wid = lax.axis_index("s") * _NC + lax.axis_index("c")
    p0 = wid * _PPW

    pltpu.sync_copy(ids_hbm, ids_v)
    pltpu.sync_copy(wp_hbm.at[pl.ds(p0, _PPW)], pos_v)
    pltpu.sync_copy(wt_hbm, wt_v)
    pltpu.sync_copy(g_hbm, g_v)
    pltpu.sync_copy(bt_hbm, bt_v)

    def fold(t, _):
        def fj(j, _c):
            sl = pl.ds(j * _L, _L)
            pos_v[t, sl] = pos_v[t, sl] + wt_v[0, sl]
            return 0
        return lax.fori_loop(0, _NVR, fj, 0)

    lax.fori_loop(0, _PPW, fold, 0)

    inv_h = 1.0 / _H
    lane_iota = lax.iota(jnp.int32, _L)
    zero = jnp.zeros((_L,), jnp.float32)

    def start_gather(b, buf, sem):
        idx = ids_v[b, pl.ds(p0, _PPW)]
        pltpu.async_copy(ww_hbm.at[idx], buf, sem)

    def wait_gather(buf, sem):
        pltpu.make_async_copy(ww_hbm.at[pl.ds(0, _PPW)], buf, sem).wait()

    def start_out(b, buf, sem):
        pltpu.async_copy(buf, out_hbm.at[pl.ds(b * _S + p0, _PPW)], sem)

    def wait_out(buf, sem):
        pltpu.make_async_copy(buf, out_hbm.at[pl.ds(0, _PPW)], sem).wait()

    def compute(buf):
        def p1(t, _):
            col = jnp.broadcast_to(t, (_L,))
            accs = [zero] * 4
            acc2s = [zero] * 4
            for j in range(_NVR):
                sl = pl.ds(j * _L, _L)
                x = buf[t, sl] + pos_v[t, sl]
                buf[t, sl] = x
                accs[j % 4] = accs[j % 4] + x
                acc2s[j % 4] = acc2s[j % 4] + x * x
            acc = (accs[0] + accs[1]) + (accs[2] + accs[3])
            acc2 = (acc2s[0] + acc2s[1]) + (acc2s[2] + acc2s[3])
            plsc.store_scatter(sums_v, [lane_iota, col], acc)
            plsc.store_scatter(sq_v, [lane_iota, col], acc2)
            return 0

        lax.fori_loop(0, _PPW, p1, 0)

        t1 = [zero] * 4
        t2 = [zero] * 4
        for i in range(_L):
            sl = pl.ds(0, _L)
            t1[i % 4] = t1[i % 4] + sums_v[i, sl]
            t2[i % 4] = t2[i % 4] + sq_v[i, sl]
        tot = (t1[0] + t1[1]) + (t1[2] + t1[3])
        tot2 = (t2[0] + t2[1]) + (t2[2] + t2[3])
        mean_vec = tot * inv_h
        var = tot2 * inv_h - mean_vec * mean_vec + _EPS
        iv = lax.bitcast_convert_type(var, jnp.int32)
        iv = 0x5F3759DF - lax.shift_right_logical(iv, 1)
        y = lax.bitcast_convert_type(iv, jnp.float32)
        y = y * (1.5 - 0.5 * var * y * y)
        y = y * (1.5 - 0.5 * var * y * y)
        y = y * (1.5 - 0.5 * var * y * y)

        means = [jnp.broadcast_to(mean_vec[t], (_L,)) for t in range(_PPW)]
        rstds = [jnp.broadcast_to(y[t], (_L,)) for t in range(_PPW)]

        def p2(j, _):
            sl = pl.ds(j * _L, _L)
            g = g_v[sl]
            bt = bt_v[sl]
            for t in range(_PPW):
                x = buf[t, sl]
                buf[t, sl] = (x - means[t]) * (rstds[t] * g) + bt
            return 0

        lax.fori_loop(0, _NVR, p2, 0)

    start_gather(0, rows_a, gsem_a)

    bufs = ((rows_a, gsem_a, osem_a), (rows_b, gsem_b, osem_b))

    def outer(i2, _):
        for k in range(2):
            b = i2 * 2 + k
            buf, gsem, osem = bufs[k]
            obuf, ogsem, oosem = bufs[1 - k]
            if k == 0:
                @pl.when(i2 > 0)
                def _():
                    wait_out(obuf, oosem)
                start_gather(b + 1, obuf, ogsem)
            else:
                wait_out(obuf, oosem)

                @pl.when(i2 < _B // 2 - 1)
                def _():
                    start_gather(b + 1, obuf, ogsem)
            wait_gather(buf, gsem)
            compute(buf)
            start_out(b, buf, osem)
        return 0

    lax.fori_loop(0, _B // 2, outer, 0)
    wait_out(rows_b, osem_b)


@jax.jit
def _launch(ids, ww, wp, wt, g, bt):
    mesh = plsc.VectorSubcoreMesh(core_axis_name="c", subcore_axis_name="s")
    run = functools.partial(
        pl.kernel,
        out_type=jax.ShapeDtypeStruct((_B * _S, _H), jnp.float32),
        mesh=mesh,
        compiler_params=pltpu.CompilerParams(needs_layout_passes=False),
        scratch_types=[
            pltpu.VMEM((_B, _S), jnp.int32),
            pltpu.VMEM((_PPW, _H), jnp.float32),
            pltpu.VMEM((2, _H), jnp.float32),
            pltpu.VMEM((_H,), jnp.float32),
            pltpu.VMEM((_H,), jnp.float32),
            pltpu.VMEM((_PPW, _H), jnp.float32),
            pltpu.VMEM((_PPW, _H), jnp.float32),
            pltpu.VMEM((_L, _PPW), jnp.float32),
            pltpu.VMEM((_L, _PPW), jnp.float32),
            pltpu.SemaphoreType.DMA,
            pltpu.SemaphoreType.DMA,
            pltpu.SemaphoreType.DMA,
            pltpu.SemaphoreType.DMA,
        ],
    )(_body)
    return run(ids, ww, wp, wt, g, bt)


def kernel(input_ids, W_word, W_pos, W_type, gamma, beta):
    ids = input_ids.astype(jnp.int32)
    out = _launch(ids, W_word, W_pos, W_type, gamma, beta)
    return out.reshape(_B, _S, _H)


# speedup vs baseline: 3.5498x; 1.0374x over previous
import functools

import jax
import jax.numpy as jnp
from jax import lax
from jax.experimental import pallas as pl
from jax.experimental.pallas import tpu as pltpu
from jax.experimental.pallas import tpu_sc as plsc

_VOCAB = 30522
_H = 768
_B = 64
_S = 512
_EPS = 1e-5

_NC = 2
_NS = 16
_NW = _NC * _NS
_PPW = _S // _NW
_L = 16
_NVR = _H // _L
_NP = _B // 2


def _body(ids_hbm, ww_hbm, wp_hbm, wt_hbm, g_hbm, bt_hbm, out_hbm,
          ids_v, pos_v, wt_v, g_v, bt_v,
          rows0a, rows0b, rows1a, rows1b,
          sums_a, sq_a, sums_b, sq_b,
          gsem0, gsem1, osem0, osem1):
    wid = lax.axis_index("s") * _NC + lax.axis_index("c")
    p0 = wid * _PPW

    pltpu.sync_copy(ids_hbm, ids_v)
    pltpu.sync_copy(wp_hbm.at[pl.ds(p0, _PPW)], pos_v)
    pltpu.sync_copy(wt_hbm, wt_v)
    pltpu.sync_copy(g_hbm, g_v)
    pltpu.sync_copy(bt_hbm, bt_v)

    def fold(t, _):
        def fj(j, _c):
            sl = pl.ds(j * _L, _L)
            pos_v[t, sl] = pos_v[t, sl] + wt_v[0, sl]
            return 0
        return lax.fori_loop(0, _NVR, fj, 0)

    lax.fori_loop(0, _PPW, fold, 0)

    inv_h = 1.0 / _H
    lane_iota = lax.iota(jnp.int32, _L)
    zero = jnp.zeros((_L,), jnp.float32)

    def start_gathers(p, bufa, bufb, sem):
        idxa = ids_v[2 * p, pl.ds(p0, _PPW)]
        pltpu.async_copy(ww_hbm.at[idxa], bufa, sem)
        idxb = ids_v[2 * p + 1, pl.ds(p0, _PPW)]
        pltpu.async_copy(ww_hbm.at[idxb], bufb, sem)

    def wait_gathers(bufa, bufb, sem):
        pltpu.make_async_copy(ww_hbm.at[pl.ds(0, _PPW)], bufa, sem).wait()
        pltpu.make_async_copy(ww_hbm.at[pl.ds(0, _PPW)], bufb, sem).wait()

    def start_outs(p, bufa, bufb, sem):
        pltpu.async_copy(bufa, out_hbm.at[pl.ds(2 * p * _S + p0, _PPW)], sem)
        pltpu.async_copy(bufb, out_hbm.at[pl.ds((2 * p + 1) * _S + p0, _PPW)], sem)

    def wait_outs(bufa, bufb, sem):
        pltpu.make_async_copy(bufa, out_hbm.at[pl.ds(0, _PPW)], sem).wait()
        pltpu.make_async_copy(bufb, out_hbm.at[pl.ds(0, _PPW)], sem).wait()

    def ln_scale(sums_ref, sq_ref):
        t1 = [zero] * 4
        t2 = [zero] * 4
        for i in range(_L):
            sl = pl.ds(0, _L)
            t1[i % 4] = t1[i % 4] + sums_ref[i, sl]
            t2[i % 4] = t2[i % 4] + sq_ref[i, sl]
        tot = (t1[0] + t1[1]) + (t1[2] + t1[3])
        tot2 = (t2[0] + t2[1]) + (t2[2] + t2[3])
        mean_vec = tot * inv_h
        var = tot2 * inv_h - mean_vec * mean_vec + _EPS
        iv = lax.bitcast_convert_type(var, jnp.int32)
        iv = 0x5F3759DF - lax.shift_right_logical(iv, 1)
        y = lax.bitcast_convert_type(iv, jnp.float32)
        y = y * (1.5 - 0.5 * var * y * y)
        y = y * (1.5 - 0.5 * var * y * y)
        y = y * (1.5 - 0.5 * var * y * y)
        return mean_vec, y

    def compute(bufa, bufb):
        def p1(t, _):
            col = jnp.broadcast_to(t, (_L,))
            acca = [zero] * 2
            acc2a = [zero] * 2
            accb = [zero] * 2
            acc2b = [zero] * 2
            for j in range(_NVR):
                sl = pl.ds(j * _L, _L)
                q = pos_v[t, sl]
                xa = bufa[t, sl] + q
                bufa[t, sl] = xa
                xb = bufb[t, sl] + q
                bufb[t, sl] = xb
                acca[j % 2] = acca[j % 2] + xa
                acc2a[j % 2] = acc2a[j % 2] + xa * xa
                accb[j % 2] = accb[j % 2] + xb
                acc2b[j % 2] = acc2b[j % 2] + xb * xb
            plsc.store_scatter(sums_a, [lane_iota, col], acca[0] + acca[1])
            plsc.store_scatter(sq_a, [lane_iota, col], acc2a[0] + acc2a[1])
            plsc.store_scatter(sums_b, [lane_iota, col], accb[0] + accb[1])
            plsc.store_scatter(sq_b, [lane_iota, col], acc2b[0] + acc2b[1])
            return 0

        lax.fori_loop(0, _PPW, p1, 0)

        mean_a, rstd_a = ln_scale(sums_a, sq_a)
        mean_b, rstd_b = ln_scale(sums_b, sq_b)

        for buf, mean_vec, rstd_vec in ((bufa, mean_a, rstd_a),
                                        (bufb, mean_b, rstd_b)):
            means = [jnp.broadcast_to(mean_vec[t], (_L,)) for t in range(_PPW)]
            rstds = [jnp.broadcast_to(rstd_vec[t], (_L,)) for t in range(_PPW)]

            def p2(j, _, buf=buf, means=means, rstds=rstds):
                sl = pl.ds(j * _L, _L)
                g = g_v[sl]
                bt = bt_v[sl]
                for t in range(_PPW):
                    x = buf[t, sl]
                    buf[t, sl] = (x - means[t]) * (rstds[t] * g) + bt
                return 0

            lax.fori_loop(0, _NVR, p2, 0)

    slots = ((rows0a, rows0b, gsem0, osem0), (rows1a, rows1b, gsem1, osem1))
    start_gathers(0, rows0a, rows0b, gsem0)

    def outer(i2, _):
        for k in range(2):
            p = i2 * 2 + k
            bufa, bufb, gsem, osem = slots[k]
            obufa, obufb, ogsem, oosem = slots[1 - k]
            if k == 0:
                @pl.when(i2 > 0)
                def _():
                    wait_outs(obufa, obufb, oosem)
                start_gathers(p + 1, obufa, obufb, ogsem)
            else:
                wait_outs(obufa, obufb, oosem)

                @pl.when(i2 < _NP // 2 - 1)
                def _():
                    start_gathers(p + 1, obufa, obufb, ogsem)
            wait_gathers(bufa, bufb, gsem)
            compute(bufa, bufb)
            start_outs(p, bufa, bufb, osem)
        return 0

    lax.fori_loop(0, _NP // 2, outer, 0)
    wait_outs(rows1a, rows1b, osem1)


@jax.jit
def _launch(ids, ww, wp, wt, g, bt):
    mesh = plsc.VectorSubcoreMesh(core_axis_name="c", subcore_axis_name="s")
    run = functools.partial(
        pl.kernel,
        out_type=jax.ShapeDtypeStruct((_B * _S, _H), jnp.float32),
        mesh=mesh,
        compiler_params=pltpu.CompilerParams(needs_layout_passes=False),
        scratch_types=[
            pltpu.VMEM((_B, _S), jnp.int32),
            pltpu.VMEM((_PPW, _H), jnp.float32),
            pltpu.VMEM((2, _H), jnp.float32),
            pltpu.VMEM((_H,), jnp.float32),
            pltpu.VMEM((_H,), jnp.float32),
            pltpu.VMEM((_PPW, _H), jnp.float32),
            pltpu.VMEM((_PPW, _H), jnp.float32),
            pltpu.VMEM((_PPW, _H), jnp.float32),
            pltpu.VMEM((_PPW, _H), jnp.float32),
            pltpu.VMEM((_L, _PPW), jnp.float32),
            pltpu.VMEM((_L, _PPW), jnp.float32),
            pltpu.VMEM((_L, _PPW), jnp.float32),
            pltpu.VMEM((_L, _PPW), jnp.float32),
            pltpu.SemaphoreType.DMA,
            pltpu.SemaphoreType.DMA,
            pltpu.SemaphoreType.DMA,
            pltpu.SemaphoreType.DMA,
        ],
    )(_body)
    return run(ids, ww, wp, wt, g, bt)


def kernel(input_ids, W_word, W_pos, W_type, gamma, beta):
    ids = input_ids.astype(jnp.int32)
    out = _launch(ids, W_word, W_pos, W_type, gamma, beta)
    return out.reshape(_B, _S, _H)
